# baseline (speedup 1.0000x reference)
import functools

import jax
import jax.numpy as jnp
from jax import lax
from jax.experimental import pallas as pl
from jax.experimental.pallas import tpu as pltpu
from jax.experimental.pallas import tpu_sc as plsc

_N = 100000
_E = 3200000
_STEPS = 4
_F = 8
_G = 16
_NRA = (_N + 96) // _G
_NC, _NS = 2, 16
_NW = _NC * _NS
_CH = 128
_K = 24
_GROUPS = 1056
_EPAD = _GROUPS * _K * _CH
_OUTER = _GROUPS // _NW
_O0, _O1 = 34, 32
_NACC = _N + 96
_RPT = _NACC // _NS


def _segsum_body(tab_hbm, src_hbm, dst_hbm, zero_hbm, out_hbm,
                 srcv0, dstv0, srcv1, dstv1, rows0, rows1,
                 acc_sh, isem, gsem, ssem):
    c = lax.axis_index("c")
    s = lax.axis_index("s")
    pltpu.sync_copy(zero_hbm, acc_sh.at[pl.ds(s * _RPT, _RPT)])
    plsc.subcore_barrier()

    nout = _O0 - (_O0 - _O1) * c
    base_g = c * _NS * _O0 + s * nout

    def crow(t):
        return pl.multiple_of((base_g + t) * _K, 8)

    def fire_idx(t, sv, dv):
        pltpu.async_copy(src_hbm.at[pl.ds(crow(t), _K)], sv, isem)
        pltpu.async_copy(dst_hbm.at[pl.ds(crow(t), _K)], dv, isem)

    def drain_idx(sv, dv):
        pltpu.make_async_copy(src_hbm.at[pl.ds(0, _K)], sv, isem).wait()
        pltpu.make_async_copy(dst_hbm.at[pl.ds(0, _K)], dv, isem).wait()

    def fire_g(sv, rows):
        for j in range(_K):
            pltpu.async_copy(tab_hbm.at[sv.at[j]], rows.at[j], gsem)

    def drain_g(rows):
        for j in range(_K):
            pltpu.make_async_copy(tab_hbm.at[pl.ds(0, _CH)], rows.at[j],
                                  gsem).wait()

    def scat(rows, dv):
        sds = [pltpu.async_copy(rows.at[j], acc_sh.at[dv.at[j]], ssem,
                                add=True)
               for j in range(_K)]
        for d in sds:
            d.wait()

    pltpu.sync_copy(src_hbm.at[pl.ds(crow(0), _K)], srcv0)
    pltpu.sync_copy(dst_hbm.at[pl.ds(crow(0), _K)], dstv0)
    fire_g(srcv0, rows0)
    fire_idx(1, srcv1, dstv1)

    @pl.loop(0, nout // 2)
    def _body(gg):
        t0 = gg * 2
        drain_idx(srcv1, dstv1)
        drain_g(rows0)
        fire_g(srcv1, rows1)
        scat(rows0, dstv0)
        fire_idx(t0 + 2, srcv0, dstv0)
        drain_g(rows1)
        scat(rows1, dstv1)
        fire_idx(t0 + 3, srcv1, dstv1)
        drain_idx(srcv0, dstv0)
        fire_g(srcv0, rows0)

    drain_g(rows0)
    drain_idx(srcv1, dstv1)

    plsc.subcore_barrier()
    pltpu.sync_copy(acc_sh.at[pl.ds(s * _RPT, _RPT)],
                    out_hbm.at[c, pl.ds(s * _RPT, _RPT)])


def _deg_body(ones_hbm, dst_hbm, zero_hbm, out_hbm,
              dstv0, dstv1, obuf, acc_sh, isem, ssem):
    c = lax.axis_index("c")
    s = lax.axis_index("s")
    wid = c * _NS + s
    pltpu.sync_copy(zero_hbm, acc_sh.at[pl.ds(s * _RPT, _RPT)])
    pltpu.sync_copy(ones_hbm, obuf)
    plsc.subcore_barrier()

    def crow(t):
        return pl.multiple_of((wid * _OUTER + t) * _K, 8)

    def scat(dv):
        sds = [pltpu.async_copy(obuf, acc_sh.at[dv.at[j]], ssem, add=True)
               for j in range(_K)]
        for d in sds:
            d.wait()

    def drain_idx(dv):
        pltpu.make_async_copy(dst_hbm.at[pl.ds(0, _K)], dv, isem).wait()

    pltpu.sync_copy(dst_hbm.at[pl.ds(crow(0), _K)], dstv0)
    pltpu.async_copy(dst_hbm.at[pl.ds(crow(1), _K)], dstv1, isem)

    @pl.loop(0, _OUTER // 2)
    def _body(gg):
        t0 = gg * 2
        scat(dstv0)
        pltpu.async_copy(dst_hbm.at[pl.ds(crow(t0 + 2), _K)], dstv0, isem)
        drain_idx(dstv1)
        scat(dstv1)
        pltpu.async_copy(dst_hbm.at[pl.ds(crow(t0 + 3), _K)], dstv1, isem)
        drain_idx(dstv0)

    scat(dstv0)
    drain_idx(dstv1)

    plsc.subcore_barrier()
    pltpu.sync_copy(acc_sh.at[pl.ds(s * _RPT, _RPT)],
                    out_hbm.at[c, pl.ds(s * _RPT, _RPT)])


_degsum = functools.partial(
    pl.kernel,
    _deg_body,
    out_type=jax.ShapeDtypeStruct((_NC, _NACC, _F), jnp.float32),
    mesh=plsc.VectorSubcoreMesh(core_axis_name="c", subcore_axis_name="s"),
    compiler_params=pltpu.CompilerParams(use_tc_tiling_on_sc=False),
    scratch_types=[
        pltpu.VMEM((_K, _CH), jnp.int32),
        pltpu.VMEM((_K, _CH), jnp.int32),
        pltpu.VMEM((_CH, _F), jnp.float32),
        pltpu.VMEM_SHARED((_NACC, _F), jnp.float32),
        pltpu.SemaphoreType.DMA,
        pltpu.SemaphoreType.DMA,
    ],
)()


_segsum = functools.partial(
    pl.kernel,
    _segsum_body,
    out_type=jax.ShapeDtypeStruct((_NC, _NACC, _F), jnp.float32),
    mesh=plsc.VectorSubcoreMesh(core_axis_name="c", subcore_axis_name="s"),
    compiler_params=pltpu.CompilerParams(use_tc_tiling_on_sc=False),
    scratch_types=[
        pltpu.VMEM((_K, _CH), jnp.int32),
        pltpu.VMEM((_K, _CH), jnp.int32),
        pltpu.VMEM((_K, _CH), jnp.int32),
        pltpu.VMEM((_K, _CH), jnp.int32),
        pltpu.VMEM((_K, _CH, _F), jnp.float32),
        pltpu.VMEM((_K, _CH, _F), jnp.float32),
        pltpu.VMEM_SHARED((_NACC, _F), jnp.float32),
        pltpu.SemaphoreType.DMA,
        pltpu.SemaphoreType.DMA,
        pltpu.SemaphoreType.DMA,
    ],
)()


_BLK = 368
_GRID = _NRA // _BLK


def _k0_body(dp0, dp1, x, wb, dis, invd, h, g):
    deg = dp0[0] + dp1[0] + 1.0
    di = lax.rsqrt(deg)
    dis[...] = di
    invd[...] = di * di
    hh = jnp.dot(x[...], wb[...], preferred_element_type=jnp.float32,
                  precision=lax.Precision.HIGHEST)
    h[...] = hh
    g[...] = di * hh


def _k1_body(ap0, ap1, h, dis, invd, b, wb, x1, hn, gn):
    x = dis[...] * (ap0[0] + ap1[0]) + invd[...] * h[...] + b[...]
    x1[...] = x
    hh = jnp.dot(x, wb[...], preferred_element_type=jnp.float32,
                  precision=lax.Precision.HIGHEST)
    hn[...] = hh
    gn[...] = dis[...] * hh


def _kmid_body(ap0, ap1, h, xres, dis, invd, msum, mask, b, lg, lb, wb, bn,
               xn, hn, gn):
    u = dis[...] * (ap0[0] + ap1[0]) + invd[...] * h[...] + b[...]
    mu = jnp.dot(u, msum[...], preferred_element_type=jnp.float32,
                  precision=lax.Precision.HIGHEST)
    d = (u - mu) * mask[...]
    var = jnp.dot(d * d, msum[...], preferred_element_type=jnp.float32,
                  precision=lax.Precision.HIGHEST)
    v = d * lax.rsqrt(var + 1e-5) * lg[...] + lb[...]
    x = jnp.maximum(v + xres[...], 0.0)
    xn[...] = x
    hh = jnp.dot(x, wb[...], preferred_element_type=jnp.float32,
                  precision=lax.Precision.HIGHEST) + bn[...]
    hn[...] = hh
    gn[...] = dis[...] * hh


def _row_spec():
    return pl.BlockSpec((_BLK, 128), lambda i: (i, 0))


def _par_spec(core):
    return pl.BlockSpec((1, _BLK, 128), lambda i, _c=core: (_c, i, 0))


def _cst_spec(shape):
    return pl.BlockSpec(shape, lambda i: (0,) * len(shape))


_ROW_OUT = jax.ShapeDtypeStruct((_NRA, 128), jnp.float32)


def _pad2(a, r, c):
    return jnp.zeros((r, c), a.dtype).at[:a.shape[0], :a.shape[1]].set(a)


def _pad1(a, n):
    return jnp.zeros((n,), a.dtype).at[:a.shape[0]].set(a)


def _bd(w):
    return jnp.kron(jnp.eye(_G, dtype=w.dtype), w)


def kernel(node, edges, W1, b1, Wc, bc, lng, lnb, fcW, fcb):
    i32 = jnp.int32
    f32 = jnp.float32
    npad = _EPAD - _E + 2 * _K * _CH
    srcp = jnp.concatenate(
        [edges[0], (jnp.arange(npad, dtype=i32) * 1237) % _N]
    ).reshape(-1, _CH)
    dstp = jnp.concatenate(
        [edges[1], jnp.full((npad,), _N, i32)]).reshape(-1, _CH)
    ones_tab = jnp.ones((_CH, _F), f32)
    zrows = jnp.zeros((_RPT, _F), f32)
    nodep = _pad2(node, _NACC, _F).reshape(_NRA, 128)
    wb1 = _bd(_pad2(W1, _F, _F).T)
    wbc = [_bd(_pad2(Wc[i], _F, _F).T) for i in range(_STEPS)]
    wfc = _bd(_pad2(fcW, _F, _F).T)
    bcp = [jnp.tile(_pad1(bc[i], _F), _G) for i in range(_STEPS)]
    lngp = [jnp.tile(_pad1(lng[i], _F), _G) for i in range(_STEPS)]
    lnbp = [jnp.tile(_pad1(lnb[i], _F), _G) for i in range(_STEPS)]
    b1p = jnp.tile(_pad1(b1, _F), _G)
    fcbp = jnp.tile(_pad1(fcb, _F), _G)
    zb = jnp.zeros((128,), f32)
    msum = _bd((jnp.arange(_F) < 6).astype(f32)[:, None]
               * jnp.full((_F, _F), 1.0 / 6.0))
    mask = jnp.tile((jnp.arange(_F) < 6).astype(f32), _G)

    dp = _degsum(ones_tab, dstp, zrows).reshape(_NC, _NRA, 128)

    dis, invd, h, g = pl.pallas_call(
        _k0_body,
        grid=(_GRID,),
        in_specs=[_par_spec(0), _par_spec(1), _row_spec(),
                  _cst_spec((128, 128))],
        out_specs=[_row_spec()] * 4,
        out_shape=[_ROW_OUT] * 4,
    )(dp, dp, nodep, wb1)

    ap = _segsum(g.reshape(_NACC, _F), srcp, dstp, zrows).reshape(_NC, _NRA, 128)
    xres, h, g = pl.pallas_call(
        _k1_body,
        grid=(_GRID,),
        in_specs=[_par_spec(0), _par_spec(1), _row_spec(),
                  _row_spec(), _row_spec(),
                  _cst_spec((128,)), _cst_spec((128, 128))],
        out_specs=[_row_spec()] * 3,
        out_shape=[_ROW_OUT] * 3,
    )(ap, ap, h, dis, invd, b1p, wbc[0])

    for i in range(_STEPS):
        ap = _segsum(g.reshape(_NACC, _F), srcp, dstp, zrows).reshape(_NC, _NRA, 128)
        last = i == _STEPS - 1
        wn = wfc if last else wbc[i + 1]
        bn = fcbp if last else zb
        xres, h, g = pl.pallas_call(
            _kmid_body,
            grid=(_GRID,),
            in_specs=[_par_spec(0), _par_spec(1), _row_spec(), _row_spec(),
                      _row_spec(), _row_spec(),
                      _cst_spec((128, 128)), _cst_spec((128,)),
                      _cst_spec((128,)), _cst_spec((128,)), _cst_spec((128,)),
                      _cst_spec((128, 128)), _cst_spec((128,))],
            out_specs=[_row_spec()] * 3,
            out_shape=[_ROW_OUT] * 3,
        )(ap, ap, h, xres, dis, invd, msum, mask, bcp[i], lngp[i], lnbp[i],
          wn, bn)

    return h.reshape(_NACC, _F)[:_N, :3]


# speedup vs baseline: 1.0353x; 1.0353x over previous
import functools

import jax
import jax.numpy as jnp
from jax import lax
from jax.experimental import pallas as pl
from jax.experimental.pallas import tpu as pltpu
from jax.experimental.pallas import tpu_sc as plsc

_N = 100000
_E = 3200000
_STEPS = 4
_F = 8
_G = 16
_NRA = (_N + 96) // _G
_NC, _NS = 2, 16
_NW = _NC * _NS
_CH = 128
_K = 16
_EPT = 100352
_EPAD = _EPT * _NW
_CPT = _EPT // _CH
_OUTER = _CPT // _K
_O0, _O1 = 50, 48
_NACC = _N + 96
_RPT = _NACC // _NS


def _segsum_body(tab_hbm, src_hbm, dst_hbm, zero_hbm, out_hbm,
                 srcv0, dstv0, srcv1, dstv1, rows0, rows1,
                 acc_sh, isem, gsem, ssem):
    c = lax.axis_index("c")
    s = lax.axis_index("s")
    pltpu.sync_copy(zero_hbm, acc_sh.at[pl.ds(s * _RPT, _RPT)])
    plsc.subcore_barrier()

    nout = _O0 - (_O0 - _O1) * c
    base_g = c * _NS * _O0 + s * nout

    def crow(t):
        return pl.multiple_of((base_g + t) * _K, 8)

    def fire_idx(t, sv, dv):
        pltpu.async_copy(src_hbm.at[pl.ds(crow(t), _K)], sv, isem)
        pltpu.async_copy(dst_hbm.at[pl.ds(crow(t), _K)], dv, isem)

    def drain_idx(sv, dv):
        pltpu.make_async_copy(src_hbm.at[pl.ds(0, _K)], sv, isem).wait()
        pltpu.make_async_copy(dst_hbm.at[pl.ds(0, _K)], dv, isem).wait()

    def fire_g(sv, rows):
        for j in range(_K):
            pltpu.async_copy(tab_hbm.at[sv.at[j]], rows.at[j], gsem)

    def drain_g(rows):
        for j in range(_K):
            pltpu.make_async_copy(tab_hbm.at[pl.ds(0, _CH)], rows.at[j],
                                  gsem).wait()

    def scat(rows, dv):
        sds = [pltpu.async_copy(rows.at[j], acc_sh.at[dv.at[j]], ssem,
                                add=True)
               for j in range(_K)]
        for d in sds:
            d.wait()

    pltpu.sync_copy(src_hbm.at[pl.ds(crow(0), _K)], srcv0)
    pltpu.sync_copy(dst_hbm.at[pl.ds(crow(0), _K)], dstv0)
    fire_g(srcv0, rows0)
    fire_idx(1, srcv1, dstv1)

    @pl.loop(0, nout // 2)
    def _body(gg):
        t0 = gg * 2
        drain_idx(srcv1, dstv1)
        drain_g(rows0)
        fire_g(srcv1, rows1)
        scat(rows0, dstv0)
        fire_idx(t0 + 2, srcv0, dstv0)
        drain_g(rows1)
        scat(rows1, dstv1)
        fire_idx(t0 + 3, srcv1, dstv1)
        drain_idx(srcv0, dstv0)
        fire_g(srcv0, rows0)

    drain_g(rows0)
    drain_idx(srcv1, dstv1)

    plsc.subcore_barrier()
    pltpu.sync_copy(acc_sh.at[pl.ds(s * _RPT, _RPT)],
                    out_hbm.at[c, pl.ds(s * _RPT, _RPT)])


def _deg_body(ones_hbm, dst_hbm, zero_hbm, out_hbm,
              dstv0, dstv1, obuf, acc_sh, isem, ssem):
    c = lax.axis_index("c")
    s = lax.axis_index("s")
    wid = c * _NS + s
    pltpu.sync_copy(zero_hbm, acc_sh.at[pl.ds(s * _RPT, _RPT)])
    pltpu.sync_copy(ones_hbm, obuf)
    plsc.subcore_barrier()

    def crow(t):
        return pl.multiple_of((wid * _OUTER + t) * _K, 8)

    def scat(dv):
        sds = [pltpu.async_copy(obuf, acc_sh.at[dv.at[j]], ssem, add=True)
               for j in range(_K)]
        for d in sds:
            d.wait()

    def drain_idx(dv):
        pltpu.make_async_copy(dst_hbm.at[pl.ds(0, _K)], dv, isem).wait()

    pltpu.sync_copy(dst_hbm.at[pl.ds(crow(0), _K)], dstv0)
    pltpu.async_copy(dst_hbm.at[pl.ds(crow(1), _K)], dstv1, isem)

    @pl.loop(0, _OUTER // 2)
    def _body(gg):
        t0 = gg * 2
        scat(dstv0)
        pltpu.async_copy(dst_hbm.at[pl.ds(crow(t0 + 2), _K)], dstv0, isem)
        drain_idx(dstv1)
        scat(dstv1)
        pltpu.async_copy(dst_hbm.at[pl.ds(crow(t0 + 3), _K)], dstv1, isem)
        drain_idx(dstv0)

    scat(dstv0)
    drain_idx(dstv1)

    plsc.subcore_barrier()
    pltpu.sync_copy(acc_sh.at[pl.ds(s * _RPT, _RPT)],
                    out_hbm.at[c, pl.ds(s * _RPT, _RPT)])


_degsum = functools.partial(
    pl.kernel,
    _deg_body,
    out_type=jax.ShapeDtypeStruct((_NC, _NACC, _F), jnp.float32),
    mesh=plsc.VectorSubcoreMesh(core_axis_name="c", subcore_axis_name="s"),
    compiler_params=pltpu.CompilerParams(use_tc_tiling_on_sc=False),
    scratch_types=[
        pltpu.VMEM((_K, _CH), jnp.int32),
        pltpu.VMEM((_K, _CH), jnp.int32),
        pltpu.VMEM((_CH, _F), jnp.float32),
        pltpu.VMEM_SHARED((_NACC, _F), jnp.float32),
        pltpu.SemaphoreType.DMA,
        pltpu.SemaphoreType.DMA,
    ],
)()


_segsum = functools.partial(
    pl.kernel,
    _segsum_body,
    out_type=jax.ShapeDtypeStruct((_NC, _NACC, _F), jnp.float32),
    mesh=plsc.VectorSubcoreMesh(core_axis_name="c", subcore_axis_name="s"),
    compiler_params=pltpu.CompilerParams(use_tc_tiling_on_sc=False),
    scratch_types=[
        pltpu.VMEM((_K, _CH), jnp.int32),
        pltpu.VMEM((_K, _CH), jnp.int32),
        pltpu.VMEM((_K, _CH), jnp.int32),
        pltpu.VMEM((_K, _CH), jnp.int32),
        pltpu.VMEM((_K, _CH, _F), jnp.float32),
        pltpu.VMEM((_K, _CH, _F), jnp.float32),
        pltpu.VMEM_SHARED((_NACC, _F), jnp.float32),
        pltpu.SemaphoreType.DMA,
        pltpu.SemaphoreType.DMA,
        pltpu.SemaphoreType.DMA,
    ],
)()


_BLK = 368
_GRID = _NRA // _BLK


def _k0_body(dp0, dp1, x, wb, dis, invd, h, g):
    deg = dp0[0] + dp1[0] + 1.0
    di = lax.rsqrt(deg)
    dis[...] = di
    invd[...] = di * di
    hh = jnp.dot(x[...], wb[...], preferred_element_type=jnp.float32,
                  precision=lax.Precision.HIGHEST)
    h[...] = hh
    g[...] = di * hh


def _k1_body(ap0, ap1, h, dis, invd, b, wb, x1, hn, gn):
    x = dis[...] * (ap0[0] + ap1[0]) + invd[...] * h[...] + b[...]
    x1[...] = x
    hh = jnp.dot(x, wb[...], preferred_element_type=jnp.float32,
                  precision=lax.Precision.HIGHEST)
    hn[...] = hh
    gn[...] = dis[...] * hh


def _kmid_body(ap0, ap1, h, xres, dis, invd, msum, mask, b, lg, lb, wb, bn,
               xn, hn, gn):
    u = dis[...] * (ap0[0] + ap1[0]) + invd[...] * h[...] + b[...]
    mu = jnp.dot(u, msum[...], preferred_element_type=jnp.float32,
                  precision=lax.Precision.HIGHEST)
    d = (u - mu) * mask[...]
    var = jnp.dot(d * d, msum[...], preferred_element_type=jnp.float32,
                  precision=lax.Precision.HIGHEST)
    v = d * lax.rsqrt(var + 1e-5) * lg[...] + lb[...]
    x = jnp.maximum(v + xres[...], 0.0)
    xn[...] = x
    hh = jnp.dot(x, wb[...], preferred_element_type=jnp.float32,
                  precision=lax.Precision.HIGHEST) + bn[...]
    hn[...] = hh
    gn[...] = dis[...] * hh


def _row_spec():
    return pl.BlockSpec((_BLK, 128), lambda i: (i, 0))


def _par_spec(core):
    return pl.BlockSpec((1, _BLK, 128), lambda i, _c=core: (_c, i, 0))


def _cst_spec(shape):
    return pl.BlockSpec(shape, lambda i: (0,) * len(shape))


_ROW_OUT = jax.ShapeDtypeStruct((_NRA, 128), jnp.float32)


def _pad2(a, r, c):
    return jnp.zeros((r, c), a.dtype).at[:a.shape[0], :a.shape[1]].set(a)


def _pad1(a, n):
    return jnp.zeros((n,), a.dtype).at[:a.shape[0]].set(a)


def _bd(w):
    return jnp.kron(jnp.eye(_G, dtype=w.dtype), w)


def kernel(node, edges, W1, b1, Wc, bc, lng, lnb, fcW, fcb):
    i32 = jnp.int32
    f32 = jnp.float32
    npad = _EPAD - _E + 2 * _K * _CH
    srcp = jnp.concatenate(
        [edges[0], (jnp.arange(npad, dtype=i32) * 1237) % _N]
    ).reshape(-1, _CH)
    dstp = jnp.concatenate(
        [edges[1], jnp.full((npad,), _N, i32)]).reshape(-1, _CH)
    ones_tab = jnp.ones((_CH, _F), f32)
    zrows = jnp.zeros((_RPT, _F), f32)
    nodep = _pad2(node, _NACC, _F).reshape(_NRA, 128)
    wb1 = _bd(_pad2(W1, _F, _F).T)
    wbc = [_bd(_pad2(Wc[i], _F, _F).T) for i in range(_STEPS)]
    wfc = _bd(_pad2(fcW, _F, _F).T)
    bcp = [jnp.tile(_pad1(bc[i], _F), _G) for i in range(_STEPS)]
    lngp = [jnp.tile(_pad1(lng[i], _F), _G) for i in range(_STEPS)]
    lnbp = [jnp.tile(_pad1(lnb[i], _F), _G) for i in range(_STEPS)]
    b1p = jnp.tile(_pad1(b1, _F), _G)
    fcbp = jnp.tile(_pad1(fcb, _F), _G)
    zb = jnp.zeros((128,), f32)
    msum = _bd((jnp.arange(_F) < 6).astype(f32)[:, None]
               * jnp.full((_F, _F), 1.0 / 6.0))
    mask = jnp.tile((jnp.arange(_F) < 6).astype(f32), _G)

    dp = _degsum(ones_tab, dstp, zrows).reshape(_NC, _NRA, 128)

    dis, invd, h, g = pl.pallas_call(
        _k0_body,
        grid=(_GRID,),
        in_specs=[_par_spec(0), _par_spec(1), _row_spec(),
                  _cst_spec((128, 128))],
        out_specs=[_row_spec()] * 4,
        out_shape=[_ROW_OUT] * 4,
    )(dp, dp, nodep, wb1)

    ap = _segsum(g.reshape(_NACC, _F), srcp, dstp, zrows).reshape(_NC, _NRA, 128)
    xres, h, g = pl.pallas_call(
        _k1_body,
        grid=(_GRID,),
        in_specs=[_par_spec(0), _par_spec(1), _row_spec(),
                  _row_spec(), _row_spec(),
                  _cst_spec((128,)), _cst_spec((128, 128))],
        out_specs=[_row_spec()] * 3,
        out_shape=[_ROW_OUT] * 3,
    )(ap, ap, h, dis, invd, b1p, wbc[0])

    for i in range(_STEPS):
        ap = _segsum(g.reshape(_NACC, _F), srcp, dstp, zrows).reshape(_NC, _NRA, 128)
        last = i == _STEPS - 1
        wn = wfc if last else wbc[i + 1]
        bn = fcbp if last else zb
        xres, h, g = pl.pallas_call(
            _kmid_body,
            grid=(_GRID,),
            in_specs=[_par_spec(0), _par_spec(1), _row_spec(), _row_spec(),
                      _row_spec(), _row_spec(),
                      _cst_spec((128, 128)), _cst_spec((128,)),
                      _cst_spec((128,)), _cst_spec((128,)), _cst_spec((128,)),
                      _cst_spec((128, 128)), _cst_spec((128,))],
            out_specs=[_row_spec()] * 3,
            out_shape=[_ROW_OUT] * 3,
        )(ap, ap, h, xres, dis, invd, msum, mask, bcp[i], lngp[i], lnbp[i],
          wn, bn)

    return h.reshape(_NACC, _F)[:_N, :3]
